# baseline (speedup 1.0000x reference)
import functools

import jax
import jax.numpy as jnp
from jax import lax
from jax.experimental import pallas as pl
from jax.experimental.pallas import tpu as pltpu
from jax.experimental.pallas import tpu_sc as plsc

VOCAB_SIZE = 30522
MASK_SELECTION_RATE = 0.15
MASK_TOKEN_ID = 103
L = 320
B = 16
S = 2048
LANES = 16
CHUNKS = S // LANES
NS = 16


def _sc_body(inputs_hbm, code_hbm, tok_out, pos_out, ids_out, w_out,
             inp_v, code_v, tok_v, pos_v, ids_v, w_v, sem_a, sem_b):
    row = lax.axis_index("s")

    cp_a = pltpu.async_copy(inputs_hbm.at[row], inp_v, sem_a)
    cp_b = pltpu.async_copy(code_hbm.at[row], code_v, sem_b)
    cp_a.wait()
    cp_b.wait()

    iota = lax.iota(jnp.int32, LANES)

    carry_vec = jnp.zeros((LANES,), jnp.int32)
    for i in range(CHUNKS):
        off = i * LANES
        tok = inp_v[pl.ds(off, LANES)]
        cd = code_v[pl.ds(off, LANES)]
        sel_raw = (cd != -1) & (tok != 0)
        inc = jnp.where(sel_raw, 1, 0).astype(jnp.int32)
        cs = plsc.cumsum(inc) + carry_vec
        selected = sel_raw & (cs <= L)
        tok_v[pl.ds(off, LANES)] = jnp.where(selected & (cd >= 0), cd, tok)
        plsc.store_scatter(pos_v, [cs - 1], off + iota, mask=selected)
        carry_vec = carry_vec + plsc.all_reduce_population_count(sel_raw)

    n_sel = jnp.minimum(carry_vec, L)

    ones_f = jnp.ones((LANES,), jnp.float32)
    zero_f = jnp.zeros((LANES,), jnp.float32)
    for i in range(L // LANES):
        sl = pl.ds(i * LANES, LANES)
        valid = (i * LANES + iota) < n_sel
        p = jnp.where(valid, pos_v[sl], 0)
        pos_v[sl] = p
        g = plsc.load_gather(inp_v, [p])
        ids_v[sl] = jnp.where(valid, g, 0)
        w_v[sl] = jnp.where(valid, ones_f, zero_f)

    st_a = pltpu.async_copy(tok_v, tok_out.at[row], sem_a)
    st_b = pltpu.async_copy(pos_v, pos_out.at[row], sem_b)
    st_a.wait()
    st_b.wait()
    st_c = pltpu.async_copy(ids_v, ids_out.at[row], sem_a)
    st_d = pltpu.async_copy(w_v, w_out.at[row], sem_b)
    st_c.wait()
    st_d.wait()


@functools.lru_cache(maxsize=1)
def _build_sc_call():
    mesh = plsc.VectorSubcoreMesh(
        core_axis_name="c", subcore_axis_name="s",
        num_cores=1, num_subcores=NS)
    return pl.kernel(
        _sc_body,
        out_type=(
            jax.ShapeDtypeStruct((B, S), jnp.int32),
            jax.ShapeDtypeStruct((B, L), jnp.int32),
            jax.ShapeDtypeStruct((B, L), jnp.int32),
            jax.ShapeDtypeStruct((B, L), jnp.float32),
        ),
        mesh=mesh,
        compiler_params=pltpu.CompilerParams(needs_layout_passes=False),
        scratch_types=[
            pltpu.VMEM((S,), jnp.int32),
            pltpu.VMEM((S,), jnp.int32),
            pltpu.VMEM((S,), jnp.int32),
            pltpu.VMEM((L,), jnp.int32),
            pltpu.VMEM((L,), jnp.int32),
            pltpu.VMEM((L,), jnp.float32),
            pltpu.SemaphoreType.DMA,
            pltpu.SemaphoreType.DMA,
        ],
    )


@functools.lru_cache(maxsize=1)
def _code_const():
    with jax.ensure_compile_time_eval():
        key = jax.random.key(42)
        k_sel, k_act, k_rand = jax.random.split(key, 3)
        u = jax.random.uniform(k_sel, (B, S))
        r = jax.random.uniform(k_act, (B, S))
        rand_tok = jax.random.randint(k_rand, (B, S), 0, VOCAB_SIZE,
                                      dtype=jnp.int32)
        code = jnp.where(
            u < MASK_SELECTION_RATE,
            jnp.where(r < 0.8, MASK_TOKEN_ID,
                      jnp.where(r < (0.8 + 0.1), rand_tok, -2)),
            -1,
        ).astype(jnp.int32)
        return jax.device_get(code)


def kernel(inputs):
    code = jnp.asarray(_code_const())
    return _build_sc_call()(inputs, code)


# speedup vs baseline: 1.2257x; 1.2257x over previous
import functools

import jax
import jax.numpy as jnp
from jax import lax
from jax.experimental import pallas as pl
from jax.experimental.pallas import tpu as pltpu
from jax.experimental.pallas import tpu_sc as plsc

VOCAB_SIZE = 30522
MASK_SELECTION_RATE = 0.15
MASK_TOKEN_ID = 103
L = 320
B = 16
S = 2048
LANES = 16
CHUNKS = S // LANES
NS = 16


def _sc_body(inputs_hbm, code_hbm, tok_out, pos_out, ids_out, w_out,
             inp_v, code_v, tok_v, pos_v, ids_v, w_v, sem_a, sem_b):
    row = lax.axis_index("s")

    cp_a = pltpu.async_copy(inputs_hbm.at[row], inp_v, sem_a)
    cp_b = pltpu.async_copy(code_hbm.at[row], code_v, sem_b)
    cp_a.wait()
    cp_b.wait()

    iota = lax.iota(jnp.int32, LANES)

    def body(i, carry_vec):
        off = i * LANES
        tok = inp_v[pl.ds(off, LANES)]
        cd = code_v[pl.ds(off, LANES)]
        sel_raw = (cd != -1) & (tok != 0)
        inc = jnp.where(sel_raw, 1, 0).astype(jnp.int32)
        cs = plsc.cumsum(inc) + carry_vec
        selected = sel_raw & (cs <= L)
        tok_v[pl.ds(off, LANES)] = jnp.where(selected & (cd >= 0), cd, tok)
        plsc.store_scatter(pos_v, [cs - 1], off + iota, mask=selected)
        return carry_vec + plsc.all_reduce_population_count(sel_raw)

    total = lax.fori_loop(0, CHUNKS, body, jnp.zeros((LANES,), jnp.int32))
    n_sel = jnp.minimum(total, L)

    ones_f = jnp.ones((LANES,), jnp.float32)
    zero_f = jnp.zeros((LANES,), jnp.float32)
    for i in range(L // LANES):
        sl = pl.ds(i * LANES, LANES)
        valid = (i * LANES + iota) < n_sel
        p = jnp.where(valid, pos_v[sl], 0)
        pos_v[sl] = p
        g = plsc.load_gather(inp_v, [p])
        ids_v[sl] = jnp.where(valid, g, 0)
        w_v[sl] = jnp.where(valid, ones_f, zero_f)

    st_a = pltpu.async_copy(tok_v, tok_out.at[row], sem_a)
    st_b = pltpu.async_copy(pos_v, pos_out.at[row], sem_b)
    st_a.wait()
    st_b.wait()
    st_c = pltpu.async_copy(ids_v, ids_out.at[row], sem_a)
    st_d = pltpu.async_copy(w_v, w_out.at[row], sem_b)
    st_c.wait()
    st_d.wait()


@functools.lru_cache(maxsize=1)
def _build_sc_call():
    mesh = plsc.VectorSubcoreMesh(
        core_axis_name="c", subcore_axis_name="s",
        num_cores=1, num_subcores=NS)
    return pl.kernel(
        _sc_body,
        out_type=(
            jax.ShapeDtypeStruct((B, S), jnp.int32),
            jax.ShapeDtypeStruct((B, L), jnp.int32),
            jax.ShapeDtypeStruct((B, L), jnp.int32),
            jax.ShapeDtypeStruct((B, L), jnp.float32),
        ),
        mesh=mesh,
        compiler_params=pltpu.CompilerParams(needs_layout_passes=False),
        scratch_types=[
            pltpu.VMEM((S,), jnp.int32),
            pltpu.VMEM((S,), jnp.int32),
            pltpu.VMEM((S,), jnp.int32),
            pltpu.VMEM((L,), jnp.int32),
            pltpu.VMEM((L,), jnp.int32),
            pltpu.VMEM((L,), jnp.float32),
            pltpu.SemaphoreType.DMA,
            pltpu.SemaphoreType.DMA,
        ],
    )


@functools.lru_cache(maxsize=1)
def _code_const():
    with jax.ensure_compile_time_eval():
        key = jax.random.key(42)
        k_sel, k_act, k_rand = jax.random.split(key, 3)
        u = jax.random.uniform(k_sel, (B, S))
        r = jax.random.uniform(k_act, (B, S))
        rand_tok = jax.random.randint(k_rand, (B, S), 0, VOCAB_SIZE,
                                      dtype=jnp.int32)
        code = jnp.where(
            u < MASK_SELECTION_RATE,
            jnp.where(r < 0.8, MASK_TOKEN_ID,
                      jnp.where(r < (0.8 + 0.1), rand_tok, -2)),
            -1,
        ).astype(jnp.int32)
        return jax.device_get(code)


def kernel(inputs):
    code = jnp.asarray(_code_const())
    return _build_sc_call()(inputs, code)


# speedup vs baseline: 1.2973x; 1.0584x over previous
import functools

import jax
import jax.numpy as jnp
from jax import lax
from jax.experimental import pallas as pl
from jax.experimental.pallas import tpu as pltpu
from jax.experimental.pallas import tpu_sc as plsc

VOCAB_SIZE = 30522
MASK_SELECTION_RATE = 0.15
MASK_TOKEN_ID = 103
L = 320
B = 16
S = 2048
LANES = 16
CHUNKS = S // LANES
NS = 16


def _sc_body(inputs_hbm, code_hbm, tok_out, pos_out, ids_out, w_out,
             inp_v, code_v, tok_v, pos_v, ids_v, w_v, sem_a, sem_b):
    row = lax.axis_index("s")

    cp_a = pltpu.async_copy(inputs_hbm.at[row], inp_v, sem_a)
    cp_b = pltpu.async_copy(code_hbm.at[row], code_v, sem_b)
    cp_a.wait()
    cp_b.wait()

    iota = lax.iota(jnp.int32, LANES)

    @plsc.parallel_loop(0, S, step=LANES, unroll=8,
                        carry=jnp.zeros((LANES,), jnp.int32))
    def total(off, carry_vec):
        tok = inp_v[pl.ds(off, LANES)]
        cd = code_v[pl.ds(off, LANES)]
        sel_raw = (cd != -1) & (tok != 0)
        inc = jnp.where(sel_raw, 1, 0).astype(jnp.int32)
        cs = plsc.cumsum(inc) + carry_vec
        selected = sel_raw & (cs <= L)
        tok_v[pl.ds(off, LANES)] = jnp.where(selected & (cd >= 0), cd, tok)
        plsc.store_scatter(pos_v, [cs - 1], off + iota, mask=selected)
        return carry_vec + plsc.all_reduce_population_count(sel_raw)

    n_sel = jnp.minimum(total, L)

    ones_f = jnp.ones((LANES,), jnp.float32)
    zero_f = jnp.zeros((LANES,), jnp.float32)
    for i in range(L // LANES):
        sl = pl.ds(i * LANES, LANES)
        valid = (i * LANES + iota) < n_sel
        p = jnp.where(valid, pos_v[sl], 0)
        pos_v[sl] = p
        g = plsc.load_gather(inp_v, [p])
        ids_v[sl] = jnp.where(valid, g, 0)
        w_v[sl] = jnp.where(valid, ones_f, zero_f)

    st_a = pltpu.async_copy(tok_v, tok_out.at[row], sem_a)
    st_b = pltpu.async_copy(pos_v, pos_out.at[row], sem_b)
    st_a.wait()
    st_b.wait()
    st_c = pltpu.async_copy(ids_v, ids_out.at[row], sem_a)
    st_d = pltpu.async_copy(w_v, w_out.at[row], sem_b)
    st_c.wait()
    st_d.wait()


@functools.lru_cache(maxsize=1)
def _build_sc_call():
    mesh = plsc.VectorSubcoreMesh(
        core_axis_name="c", subcore_axis_name="s",
        num_cores=1, num_subcores=NS)
    return pl.kernel(
        _sc_body,
        out_type=(
            jax.ShapeDtypeStruct((B, S), jnp.int32),
            jax.ShapeDtypeStruct((B, L), jnp.int32),
            jax.ShapeDtypeStruct((B, L), jnp.int32),
            jax.ShapeDtypeStruct((B, L), jnp.float32),
        ),
        mesh=mesh,
        compiler_params=pltpu.CompilerParams(needs_layout_passes=False),
        scratch_types=[
            pltpu.VMEM((S,), jnp.int32),
            pltpu.VMEM((S,), jnp.int32),
            pltpu.VMEM((S,), jnp.int32),
            pltpu.VMEM((L,), jnp.int32),
            pltpu.VMEM((L,), jnp.int32),
            pltpu.VMEM((L,), jnp.float32),
            pltpu.SemaphoreType.DMA,
            pltpu.SemaphoreType.DMA,
        ],
    )


@functools.lru_cache(maxsize=1)
def _code_const():
    with jax.ensure_compile_time_eval():
        key = jax.random.key(42)
        k_sel, k_act, k_rand = jax.random.split(key, 3)
        u = jax.random.uniform(k_sel, (B, S))
        r = jax.random.uniform(k_act, (B, S))
        rand_tok = jax.random.randint(k_rand, (B, S), 0, VOCAB_SIZE,
                                      dtype=jnp.int32)
        code = jnp.where(
            u < MASK_SELECTION_RATE,
            jnp.where(r < 0.8, MASK_TOKEN_ID,
                      jnp.where(r < (0.8 + 0.1), rand_tok, -2)),
            -1,
        ).astype(jnp.int32)
        return jax.device_get(code)


def kernel(inputs):
    code = jnp.asarray(_code_const())
    return _build_sc_call()(inputs, code)


# speedup vs baseline: 1.3043x; 1.0054x over previous
import functools

import jax
import jax.numpy as jnp
from jax import lax
from jax.experimental import pallas as pl
from jax.experimental.pallas import tpu as pltpu
from jax.experimental.pallas import tpu_sc as plsc

VOCAB_SIZE = 30522
MASK_SELECTION_RATE = 0.15
MASK_TOKEN_ID = 103
L = 320
B = 16
S = 2048
LANES = 16
CHUNKS = S // LANES
NS = 16


def _sc_body(inputs_hbm, code_hbm, tok_out, pos_out, ids_out, w_out,
             inp_v, code_v, tok_v, pos_v, ids_v, w_v, sem_a, sem_b):
    row = lax.axis_index("s")

    cp_a = pltpu.async_copy(inputs_hbm.at[row], inp_v, sem_a)
    cp_b = pltpu.async_copy(code_hbm.at[row], code_v, sem_b)
    cp_a.wait()
    cp_b.wait()

    iota = lax.iota(jnp.int32, LANES)

    @plsc.parallel_loop(0, S, step=LANES, unroll=16,
                        carry=jnp.zeros((LANES,), jnp.int32))
    def total(off, carry_vec):
        tok = inp_v[pl.ds(off, LANES)]
        cd = code_v[pl.ds(off, LANES)]
        sel_raw = (cd != -1) & (tok != 0)
        inc = jnp.where(sel_raw, 1, 0).astype(jnp.int32)
        cs = plsc.cumsum(inc) + carry_vec
        selected = sel_raw & (cs <= L)
        tok_v[pl.ds(off, LANES)] = jnp.where(selected & (cd >= 0), cd, tok)
        plsc.store_scatter(pos_v, [cs - 1], off + iota, mask=selected)
        return carry_vec + plsc.all_reduce_population_count(sel_raw)

    n_sel = jnp.minimum(total, L)

    ones_f = jnp.ones((LANES,), jnp.float32)
    zero_f = jnp.zeros((LANES,), jnp.float32)
    for i in range(L // LANES):
        sl = pl.ds(i * LANES, LANES)
        valid = (i * LANES + iota) < n_sel
        p = jnp.where(valid, pos_v[sl], 0)
        pos_v[sl] = p
        g = plsc.load_gather(inp_v, [p])
        ids_v[sl] = jnp.where(valid, g, 0)
        w_v[sl] = jnp.where(valid, ones_f, zero_f)

    st_a = pltpu.async_copy(tok_v, tok_out.at[row], sem_a)
    st_b = pltpu.async_copy(pos_v, pos_out.at[row], sem_b)
    st_a.wait()
    st_b.wait()
    st_c = pltpu.async_copy(ids_v, ids_out.at[row], sem_a)
    st_d = pltpu.async_copy(w_v, w_out.at[row], sem_b)
    st_c.wait()
    st_d.wait()


@functools.lru_cache(maxsize=1)
def _build_sc_call():
    mesh = plsc.VectorSubcoreMesh(
        core_axis_name="c", subcore_axis_name="s",
        num_cores=1, num_subcores=NS)
    return pl.kernel(
        _sc_body,
        out_type=(
            jax.ShapeDtypeStruct((B, S), jnp.int32),
            jax.ShapeDtypeStruct((B, L), jnp.int32),
            jax.ShapeDtypeStruct((B, L), jnp.int32),
            jax.ShapeDtypeStruct((B, L), jnp.float32),
        ),
        mesh=mesh,
        compiler_params=pltpu.CompilerParams(needs_layout_passes=False),
        scratch_types=[
            pltpu.VMEM((S,), jnp.int32),
            pltpu.VMEM((S,), jnp.int32),
            pltpu.VMEM((S,), jnp.int32),
            pltpu.VMEM((L,), jnp.int32),
            pltpu.VMEM((L,), jnp.int32),
            pltpu.VMEM((L,), jnp.float32),
            pltpu.SemaphoreType.DMA,
            pltpu.SemaphoreType.DMA,
        ],
    )


@functools.lru_cache(maxsize=1)
def _code_const():
    with jax.ensure_compile_time_eval():
        key = jax.random.key(42)
        k_sel, k_act, k_rand = jax.random.split(key, 3)
        u = jax.random.uniform(k_sel, (B, S))
        r = jax.random.uniform(k_act, (B, S))
        rand_tok = jax.random.randint(k_rand, (B, S), 0, VOCAB_SIZE,
                                      dtype=jnp.int32)
        code = jnp.where(
            u < MASK_SELECTION_RATE,
            jnp.where(r < 0.8, MASK_TOKEN_ID,
                      jnp.where(r < (0.8 + 0.1), rand_tok, -2)),
            -1,
        ).astype(jnp.int32)
        return jax.device_get(code)


def kernel(inputs):
    code = jnp.asarray(_code_const())
    return _build_sc_call()(inputs, code)
